# baseline (speedup 1.0000x reference)
import functools

import jax
import jax.numpy as jnp
from jax import lax
from jax.experimental import pallas as pl
from jax.experimental.pallas import tpu as pltpu
from jax.experimental.pallas import tpu_sc as plsc

N_NODES = 10000
N_EDGES = 320000
D = 128
R = 8
HID = 256
B = 1024

NC, NS, L = 2, 16, 16
NW = NC * NS
EPT = N_EDGES // NW
CW = 80
NCHUNK = EPT // CW
NBUF = 3
CNT_ROWS = 640
ACC_ROWS = N_NODES + L
ZPT = ACC_ROWS // NS
ROWS_PT = N_NODES // NS
GPT = 2 * B // NW

_mesh = plsc.VectorSubcoreMesh(core_axis_name="c", subcore_axis_name="s")
_sc_params = pltpu.CompilerParams(use_tc_tiling_on_sc=False,
                                  needs_layout_passes=False)


def _bcast_lane(v, e):
    idx = jnp.full((L,), e, jnp.int32)
    return lax.gather(
        v, idx[:, None],
        lax.GatherDimensionNumbers(offset_dims=(), collapsed_slice_dims=(0,),
                                   start_index_map=(0,)),
        (1,), mode=lax.GatherScatterMode.PROMISE_IN_BOUNDS)


@functools.partial(
    pl.kernel,
    out_type=jax.ShapeDtypeStruct((NW, CNT_ROWS, D), jnp.float32),
    mesh=_mesh,
    compiler_params=_sc_params,
    scratch_types=[
        pltpu.VMEM((CNT_ROWS, D), jnp.float32),
        pltpu.VMEM((EPT,), jnp.int32),
    ],
)
def _sc_count(edata, out, cnt_v, ebuf):
    wid = lax.axis_index("s") * NC + lax.axis_index("c")
    zeros = jnp.zeros((L,), jnp.float32)
    ones = jnp.ones((L,), jnp.float32)

    def zrow(i, _):
        for j in range(D // L):
            cnt_v[i, pl.ds(j * L, L)] = zeros
        return _
    lax.fori_loop(0, CNT_ROWS, zrow, None)

    pltpu.sync_copy(edata.at[pl.ds(wid * EPT, EPT)], ebuf)

    def cbody(g, _):
        ev = ebuf[pl.ds(g * L, L)]
        flat = lax.bitwise_and(ev, 0x1FFFF)
        row = lax.shift_right_logical(flat, 7)
        col = lax.bitwise_and(flat, 127)
        plsc.addupdate_scatter(cnt_v, [row, col], ones)
        return _
    lax.fori_loop(0, EPT // L, cbody, None)
    pltpu.sync_copy(cnt_v, out.at[wid])


@functools.partial(
    pl.kernel,
    out_type=jax.ShapeDtypeStruct((NC, N_NODES, D), jnp.float32),
    mesh=_mesh,
    compiler_params=_sc_params,
    scratch_types=[
        pltpu.VMEM((EPT,), jnp.int32),
        pltpu.VMEM((NBUF, CW), jnp.int32),
        pltpu.VMEM((NBUF, CW), jnp.int32),
        pltpu.VMEM((NBUF, CW), jnp.int32),
        pltpu.VMEM((NBUF, CW), jnp.float32),
        pltpu.VMEM((NBUF, CW, D), jnp.float32),
        pltpu.VMEM_SHARED((ACC_ROWS, D), jnp.float32),
        pltpu.SemaphoreType.DMA,
        pltpu.SemaphoreType.DMA,
    ],
)
def _sc_agg(edata, xp, w_hbm, out, ebuf, rix, wix, six, wbuf, rows, acc_sh,
            gsem, ssem):
    cid = lax.axis_index("c")
    sid = lax.axis_index("s")
    wid = sid * NC + cid
    zeros = jnp.zeros((L,), jnp.float32)

    pltpu.sync_copy(edata.at[pl.ds(wid * EPT, EPT)], ebuf)

    def zrow(i, _):
        for j in range(D // L):
            rows[0, i, pl.ds(j * L, L)] = zeros
        return _
    lax.fori_loop(0, CW, zrow, None)
    zbase = sid * ZPT
    for k in range(ZPT // CW):
        pltpu.sync_copy(rows.at[0], acc_sh.at[pl.ds(zbase + k * CW, CW)])
    ztail = ZPT % CW
    pltpu.sync_copy(rows.at[0].at[pl.ds(0, ztail)],
                    acc_sh.at[pl.ds(zbase + (ZPT // CW) * CW, ztail)])
    plsc.subcore_barrier()

    junk = jnp.full((L,), N_NODES, jnp.int32)
    for t in range(NBUF):
        for j in range(CW // L):
            six[t, pl.ds(j * L, L)] = junk

    def build_idx(c, t):
        for j in range(CW // L):
            ev = ebuf[pl.ds(c * CW + j * L, L)]
            wix[t, pl.ds(j * L, L)] = lax.bitwise_and(ev, 0x1FFFF)
            rix[t, pl.ds(j * L, L)] = (
                lax.bitwise_and(ev, 7) * N_NODES
                + lax.shift_right_logical(ev, 17))
            six[t, pl.ds(j * L, L)] = lax.bitwise_and(
                lax.shift_right_logical(ev, 3), 0x3FFF)

    def fire_g(t):
        dr = pltpu.async_copy(xp.at[rix.at[t]], rows.at[t], gsem)
        dw = pltpu.async_copy(w_hbm.at[wix.at[t]], wbuf.at[t], gsem)
        return dr, dw

    def wait_s(t):
        pltpu.make_async_copy(rows.at[t], acc_sh.at[six.at[t]], ssem).wait()

    def scale_and_scatter(t):
        for j in range(CW // L):
            wv = wbuf[t, pl.ds(j * L, L)]
            for e in range(L):
                wb = _bcast_lane(wv, e)
                r = j * L + e
                for f in range(D // L):
                    rows[t, r, pl.ds(f * L, L)] = \
                        rows[t, r, pl.ds(f * L, L)] * wb
        pltpu.async_copy(rows.at[t], acc_sh.at[six.at[t]], ssem, add=True)

    for t in range(NBUF):
        pltpu.async_copy(rows.at[t], acc_sh.at[six.at[t]], ssem, add=True)

    def pbody(k, _):
        c0 = k * NBUF
        ds = []
        for t in range(NBUF):
            wait_s(t)
            build_idx(c0 + t, t)
            ds.append(fire_g(t))
        for t in range(NBUF):
            ds[t][0].wait()
            ds[t][1].wait()
            scale_and_scatter(t)
        return _
    nsteady = NCHUNK // NBUF
    lax.fori_loop(0, nsteady, pbody, None)
    for c in range(nsteady * NBUF, NCHUNK):
        t = c - nsteady * NBUF
        wait_s(t)
        build_idx(c, t)
        dr, dw = fire_g(t)
        dr.wait()
        dw.wait()
        scale_and_scatter(t)
    for t in range(NBUF):
        wait_s(t)

    plsc.subcore_barrier()
    rbase = sid * ROWS_PT
    pltpu.sync_copy(acc_sh.at[pl.ds(rbase, ROWS_PT)],
                    out.at[cid, pl.ds(rbase, ROWS_PT)])


@functools.partial(
    pl.kernel,
    out_type=jax.ShapeDtypeStruct((2 * B, D), jnp.float32),
    mesh=_mesh,
    compiler_params=_sc_params,
    scratch_types=[
        pltpu.VMEM((GPT,), jnp.int32),
        pltpu.VMEM((GPT, D), jnp.float32),
        pltpu.SemaphoreType.DMA,
    ],
)
def _sc_gather(reps, idx_hbm, out, ibuf, robuf, sem):
    wid = lax.axis_index("s") * NC + lax.axis_index("c")
    base = wid * GPT
    pltpu.sync_copy(idx_hbm.at[pl.ds(base, GPT)], ibuf)
    pltpu.async_copy(reps.at[ibuf], robuf, sem).wait()
    pltpu.sync_copy(robuf, out.at[pl.ds(base, GPT)])


_TC_ROWS = 1000
_EB = N_EDGES // D


def _tc_pack_body(s_ref, d_ref, t_ref, out_ref):
    out_ref[...] = ((s_ref[...] << 17) | (d_ref[...] << 3) | t_ref[...])


def _tc_pack(src, dst, ty):
    return pl.pallas_call(
        _tc_pack_body,
        grid=(1,),
        in_specs=[pl.BlockSpec((_EB, D), lambda i: (i, 0))] * 3,
        out_specs=pl.BlockSpec((_EB, D), lambda i: (i, 0)),
        out_shape=jax.ShapeDtypeStruct((N_EDGES // D, D), jnp.int32),
    )(src, dst, ty)


def _tc_root_body(x_ref, wr_ref, b_ref, root_ref):
    root_ref[...] = jnp.dot(x_ref[...], wr_ref[...],
                            preferred_element_type=jnp.float32) + b_ref[...]


def _tc_root(x, w_root, b):
    return pl.pallas_call(
        _tc_root_body,
        grid=(N_NODES // _TC_ROWS,),
        in_specs=[
            pl.BlockSpec((_TC_ROWS, D), lambda i: (i, 0)),
            pl.BlockSpec((D, D), lambda i: (0, 0)),
            pl.BlockSpec((1, D), lambda i: (0, 0)),
        ],
        out_specs=pl.BlockSpec((_TC_ROWS, D), lambda i: (i, 0)),
        out_shape=jax.ShapeDtypeStruct((N_NODES, D), jnp.float32),
    )(x, w_root, b)


def _tc_xp_body(x_ref, wrel_ref, xp_ref):
    xp_ref[...] = jnp.dot(x_ref[...], wrel_ref[0],
                          preferred_element_type=jnp.float32)


def _tc_xp(x, w_rel):
    return pl.pallas_call(
        _tc_xp_body,
        grid=(R, N_NODES // _TC_ROWS),
        in_specs=[
            pl.BlockSpec((_TC_ROWS, D), lambda r, i: (i, 0)),
            pl.BlockSpec((1, D, D), lambda r, i: (r, 0, 0)),
        ],
        out_specs=pl.BlockSpec((_TC_ROWS, D),
                               lambda r, i: (r * (N_NODES // _TC_ROWS) + i, 0)),
        out_shape=jax.ShapeDtypeStruct((R * N_NODES, D), jnp.float32),
    )(x, w_rel)


def _tc_weights_body(cnt_ref, w_ref):
    total = jnp.sum(cnt_ref[...], axis=0)
    w_ref[...] = 1.0 / jnp.maximum(total, 1.0)


def _tc_weights(cnt_parts):
    blk = CNT_ROWS // 10
    return pl.pallas_call(
        _tc_weights_body,
        grid=(10,),
        in_specs=[pl.BlockSpec((NW, blk, D), lambda i: (0, i, 0))],
        out_specs=pl.BlockSpec((blk, D), lambda i: (i, 0)),
        out_shape=jax.ShapeDtypeStruct((CNT_ROWS, D), jnp.float32),
    )(cnt_parts)


def _tc_relu_body(root_ref, acc_ref, out_ref):
    out_ref[...] = jnp.maximum(root_ref[...] + acc_ref[0] + acc_ref[1], 0.0)


def _tc_relu(root, accs):
    return pl.pallas_call(
        _tc_relu_body,
        grid=(N_NODES // _TC_ROWS,),
        in_specs=[
            pl.BlockSpec((_TC_ROWS, D), lambda i: (i, 0)),
            pl.BlockSpec((NC, _TC_ROWS, D), lambda i: (0, i, 0)),
        ],
        out_specs=pl.BlockSpec((_TC_ROWS, D), lambda i: (i, 0)),
        out_shape=jax.ShapeDtypeStruct((N_NODES, D), jnp.float32),
    )(root, accs)


def _tc_head_body(g_ref, wn_ref, wf_ref, wd_ref, bf_ref, out_ref):
    g = g_ref[...]
    ne = g[:B]
    fe = g[B:]
    diff = fe - ne
    dist = jnp.sqrt(jnp.sum(diff * diff, axis=1, keepdims=True))
    h = (jnp.dot(ne, wn_ref[...], preferred_element_type=jnp.float32)
         + jnp.dot(fe, wf_ref[...], preferred_element_type=jnp.float32)
         + dist * wd_ref[...] + bf_ref[...])
    out_ref[...] = jnp.tanh(h)


def _tc_head(gat, wn, wf, wd, bf):
    return pl.pallas_call(
        _tc_head_body,
        grid=(1,),
        in_specs=[
            pl.BlockSpec((2 * B, D), lambda i: (0, 0)),
            pl.BlockSpec((D, HID), lambda i: (0, 0)),
            pl.BlockSpec((D, HID), lambda i: (0, 0)),
            pl.BlockSpec((1, HID), lambda i: (0, 0)),
            pl.BlockSpec((1, HID), lambda i: (0, 0)),
        ],
        out_specs=pl.BlockSpec((B, HID), lambda i: (0, 0)),
        out_shape=jax.ShapeDtypeStruct((B, HID), jnp.float32),
    )(gat, wn, wf, wd, bf)


def kernel(x, edge_index, edge_type, nest_tensor, food_tensor, W_root, W_rel,
           b_rgcn, W_fc, b_fc):
    eshape = (N_EDGES // D, D)
    edata = _tc_pack(edge_index[0].reshape(eshape),
                     edge_index[1].reshape(eshape),
                     edge_type.astype(jnp.int32).reshape(eshape))
    edata = edata.reshape(N_EDGES)

    root = _tc_root(x, W_root, b_rgcn.reshape(1, D))
    xp = _tc_xp(x, W_rel)

    cnt_parts = _sc_count(edata)
    w_tab = _tc_weights(cnt_parts).reshape(CNT_ROWS * D)
    accs = _sc_agg(edata, xp, w_tab)
    reps = _tc_relu(root, accs)

    gidx = jnp.concatenate([nest_tensor, food_tensor]).astype(jnp.int32)
    gat = _sc_gather(reps, gidx)

    wn = W_fc[:D]
    wf = W_fc[D:2 * D]
    wd = W_fc[2 * D].reshape(1, HID)
    return _tc_head(gat, wn, wf, wd, b_fc.reshape(1, HID))


# speedup vs baseline: 1.2667x; 1.2667x over previous
import functools

import jax
import jax.numpy as jnp
from jax import lax
from jax.experimental import pallas as pl
from jax.experimental.pallas import tpu as pltpu
from jax.experimental.pallas import tpu_sc as plsc

N_NODES = 10000
N_EDGES = 320000
D = 128
R = 8
HID = 256
B = 1024

NC, NS, L = 2, 16, 16
NW = NC * NS
EPT = N_EDGES // NW
CW = 80
NCHUNK = EPT // CW
NBUF = 2
CNT_ROWS = 640
ACC_ROWS = N_NODES + L
ZPT = ACC_ROWS // NS
ROWS_PT = N_NODES // NS
GPT = 2 * B // NW

_mesh = plsc.VectorSubcoreMesh(core_axis_name="c", subcore_axis_name="s")
_sc_params = pltpu.CompilerParams(use_tc_tiling_on_sc=False,
                                  needs_layout_passes=False)


def _bcast_lane(v, e):
    idx = jnp.full((L,), e, jnp.int32)
    return lax.gather(
        v, idx[:, None],
        lax.GatherDimensionNumbers(offset_dims=(), collapsed_slice_dims=(0,),
                                   start_index_map=(0,)),
        (1,), mode=lax.GatherScatterMode.PROMISE_IN_BOUNDS)


@functools.partial(
    pl.kernel,
    out_type=jax.ShapeDtypeStruct((NW, CNT_ROWS, D), jnp.float32),
    mesh=_mesh,
    compiler_params=_sc_params,
    scratch_types=[
        pltpu.VMEM((CNT_ROWS, D), jnp.float32),
        pltpu.VMEM((EPT,), jnp.int32),
    ],
)
def _sc_count(edata, out, cnt_v, ebuf):
    wid = lax.axis_index("s") * NC + lax.axis_index("c")
    zeros = jnp.zeros((L,), jnp.float32)
    ones = jnp.ones((L,), jnp.float32)

    def zrow(i, _):
        for j in range(D // L):
            cnt_v[i, pl.ds(j * L, L)] = zeros
        return _
    lax.fori_loop(0, CNT_ROWS, zrow, None)

    pltpu.sync_copy(edata.at[pl.ds(wid * EPT, EPT)], ebuf)

    def cbody(g, _):
        ev = ebuf[pl.ds(g * L, L)]
        flat = lax.bitwise_and(ev, 0x1FFFF)
        row = lax.shift_right_logical(flat, 7)
        col = lax.bitwise_and(flat, 127)
        plsc.addupdate_scatter(cnt_v, [row, col], ones)
        return _
    lax.fori_loop(0, EPT // L, cbody, None)
    pltpu.sync_copy(cnt_v, out.at[wid])


@functools.partial(
    pl.kernel,
    out_type=jax.ShapeDtypeStruct((NC, N_NODES, D), jnp.float32),
    mesh=_mesh,
    compiler_params=_sc_params,
    scratch_types=[
        pltpu.VMEM((EPT,), jnp.int32),
        pltpu.VMEM((NBUF, CW), jnp.int32),
        pltpu.VMEM((NBUF, CW), jnp.int32),
        pltpu.VMEM((NBUF, CW), jnp.int32),
        pltpu.VMEM((CW,), jnp.int32),
        pltpu.VMEM((NBUF, CW), jnp.float32),
        pltpu.VMEM((NBUF, CW, D), jnp.float32),
        pltpu.VMEM((CW, D), jnp.float32),
        pltpu.VMEM_SHARED((ACC_ROWS, D), jnp.float32),
        pltpu.SemaphoreType.DMA,
        pltpu.SemaphoreType.DMA,
    ],
)
def _sc_agg(edata, xp, w_hbm, out, ebuf, rix, wix, six, ssix, wbuf, rows,
            stage, acc_sh, gsem, ssem):
    cid = lax.axis_index("c")
    sid = lax.axis_index("s")
    wid = sid * NC + cid
    zeros = jnp.zeros((L,), jnp.float32)
    junk = jnp.full((L,), N_NODES, jnp.int32)

    pltpu.sync_copy(edata.at[pl.ds(wid * EPT, EPT)], ebuf)

    def zrow(i, _):
        for j in range(D // L):
            stage[i, pl.ds(j * L, L)] = zeros
        return _
    lax.fori_loop(0, CW, zrow, None)
    zbase = sid * ZPT
    for k in range(ZPT // CW):
        pltpu.sync_copy(stage, acc_sh.at[pl.ds(zbase + k * CW, CW)])
    ztail = ZPT % CW
    pltpu.sync_copy(stage.at[pl.ds(0, ztail)],
                    acc_sh.at[pl.ds(zbase + (ZPT // CW) * CW, ztail)])
    plsc.subcore_barrier()

    for j in range(CW // L):
        ssix[pl.ds(j * L, L)] = junk

    def build_idx(c, t):
        valid = c < NCHUNK
        base = jnp.where(valid, c, 0) * CW
        for j in range(CW // L):
            ev = ebuf[pl.ds(base + j * L, L)]
            wix[t, pl.ds(j * L, L)] = lax.bitwise_and(ev, 0x1FFFF)
            rix[t, pl.ds(j * L, L)] = (
                lax.bitwise_and(ev, 7) * N_NODES
                + lax.shift_right_logical(ev, 17))
            s = lax.bitwise_and(lax.shift_right_logical(ev, 3), 0x3FFF)
            six[t, pl.ds(j * L, L)] = jnp.where(valid, s, junk)

    def fire_g(t):
        pltpu.async_copy(xp.at[rix.at[t]], rows.at[t], gsem)
        pltpu.async_copy(w_hbm.at[wix.at[t]], wbuf.at[t], gsem)

    def wait_g(t):
        pltpu.make_async_copy(xp.at[rix.at[t]], rows.at[t], gsem).wait()
        pltpu.make_async_copy(w_hbm.at[wix.at[t]], wbuf.at[t], gsem).wait()

    def wait_s():
        pltpu.make_async_copy(stage, acc_sh.at[ssix], ssem).wait()

    def process(t):
        wait_g(t)
        wait_s()
        for j in range(CW // L):
            ssix[pl.ds(j * L, L)] = six[t, pl.ds(j * L, L)]
        for j in range(CW // L):
            wv = wbuf[t, pl.ds(j * L, L)]
            for e in range(L):
                wb = _bcast_lane(wv, e)
                r = j * L + e
                for f in range(D // L):
                    stage[r, pl.ds(f * L, L)] = \
                        rows[t, r, pl.ds(f * L, L)] * wb
        pltpu.async_copy(stage, acc_sh.at[ssix], ssem, add=True)

    pltpu.async_copy(stage, acc_sh.at[ssix], ssem, add=True)
    for t in range(NBUF):
        build_idx(t, t)
        fire_g(t)

    def pbody(k, _):
        c0 = k * NBUF
        for t in range(NBUF):
            process(t)
            build_idx(c0 + t + NBUF, t)
            fire_g(t)
        return _
    nsteady = NCHUNK // NBUF
    ntail = NCHUNK - nsteady * NBUF
    lax.fori_loop(0, nsteady, pbody, None)
    for t in range(ntail):
        process(t)
    for t in range(ntail, NBUF):
        wait_g(t)
    wait_s()

    plsc.subcore_barrier()
    rbase = sid * ROWS_PT
    pltpu.sync_copy(acc_sh.at[pl.ds(rbase, ROWS_PT)],
                    out.at[cid, pl.ds(rbase, ROWS_PT)])


@functools.partial(
    pl.kernel,
    out_type=jax.ShapeDtypeStruct((2 * B, D), jnp.float32),
    mesh=_mesh,
    compiler_params=_sc_params,
    scratch_types=[
        pltpu.VMEM((GPT,), jnp.int32),
        pltpu.VMEM((GPT, D), jnp.float32),
        pltpu.SemaphoreType.DMA,
    ],
)
def _sc_gather(reps, idx_hbm, out, ibuf, robuf, sem):
    wid = lax.axis_index("s") * NC + lax.axis_index("c")
    base = wid * GPT
    pltpu.sync_copy(idx_hbm.at[pl.ds(base, GPT)], ibuf)
    pltpu.async_copy(reps.at[ibuf], robuf, sem).wait()
    pltpu.sync_copy(robuf, out.at[pl.ds(base, GPT)])


_TC_ROWS = 1000
_EB = N_EDGES // D


def _tc_pack_body(s_ref, d_ref, t_ref, out_ref):
    out_ref[...] = ((s_ref[...] << 17) | (d_ref[...] << 3) | t_ref[...])


def _tc_pack(src, dst, ty):
    return pl.pallas_call(
        _tc_pack_body,
        grid=(1,),
        in_specs=[pl.BlockSpec((_EB, D), lambda i: (i, 0))] * 3,
        out_specs=pl.BlockSpec((_EB, D), lambda i: (i, 0)),
        out_shape=jax.ShapeDtypeStruct((N_EDGES // D, D), jnp.int32),
    )(src, dst, ty)


def _tc_root_body(x_ref, wr_ref, b_ref, root_ref):
    root_ref[...] = jnp.dot(x_ref[...], wr_ref[...],
                            preferred_element_type=jnp.float32) + b_ref[...]


def _tc_root(x, w_root, b):
    return pl.pallas_call(
        _tc_root_body,
        grid=(N_NODES // _TC_ROWS,),
        in_specs=[
            pl.BlockSpec((_TC_ROWS, D), lambda i: (i, 0)),
            pl.BlockSpec((D, D), lambda i: (0, 0)),
            pl.BlockSpec((1, D), lambda i: (0, 0)),
        ],
        out_specs=pl.BlockSpec((_TC_ROWS, D), lambda i: (i, 0)),
        out_shape=jax.ShapeDtypeStruct((N_NODES, D), jnp.float32),
    )(x, w_root, b)


def _tc_xp_body(x_ref, wrel_ref, xp_ref):
    xp_ref[...] = jnp.dot(x_ref[...], wrel_ref[0],
                          preferred_element_type=jnp.float32)


def _tc_xp(x, w_rel):
    return pl.pallas_call(
        _tc_xp_body,
        grid=(R, N_NODES // _TC_ROWS),
        in_specs=[
            pl.BlockSpec((_TC_ROWS, D), lambda r, i: (i, 0)),
            pl.BlockSpec((1, D, D), lambda r, i: (r, 0, 0)),
        ],
        out_specs=pl.BlockSpec((_TC_ROWS, D),
                               lambda r, i: (r * (N_NODES // _TC_ROWS) + i, 0)),
        out_shape=jax.ShapeDtypeStruct((R * N_NODES, D), jnp.float32),
    )(x, w_rel)


def _tc_weights_body(cnt_ref, w_ref):
    total = jnp.sum(cnt_ref[...], axis=0)
    w_ref[...] = 1.0 / jnp.maximum(total, 1.0)


def _tc_weights(cnt_parts):
    blk = CNT_ROWS // 10
    return pl.pallas_call(
        _tc_weights_body,
        grid=(10,),
        in_specs=[pl.BlockSpec((NW, blk, D), lambda i: (0, i, 0))],
        out_specs=pl.BlockSpec((blk, D), lambda i: (i, 0)),
        out_shape=jax.ShapeDtypeStruct((CNT_ROWS, D), jnp.float32),
    )(cnt_parts)


def _tc_relu_body(root_ref, acc_ref, out_ref):
    out_ref[...] = jnp.maximum(root_ref[...] + acc_ref[0] + acc_ref[1], 0.0)


def _tc_relu(root, accs):
    return pl.pallas_call(
        _tc_relu_body,
        grid=(N_NODES // _TC_ROWS,),
        in_specs=[
            pl.BlockSpec((_TC_ROWS, D), lambda i: (i, 0)),
            pl.BlockSpec((NC, _TC_ROWS, D), lambda i: (0, i, 0)),
        ],
        out_specs=pl.BlockSpec((_TC_ROWS, D), lambda i: (i, 0)),
        out_shape=jax.ShapeDtypeStruct((N_NODES, D), jnp.float32),
    )(root, accs)


def _tc_head_body(g_ref, wn_ref, wf_ref, wd_ref, bf_ref, out_ref):
    g = g_ref[...]
    ne = g[:B]
    fe = g[B:]
    diff = fe - ne
    dist = jnp.sqrt(jnp.sum(diff * diff, axis=1, keepdims=True))
    h = (jnp.dot(ne, wn_ref[...], preferred_element_type=jnp.float32)
         + jnp.dot(fe, wf_ref[...], preferred_element_type=jnp.float32)
         + dist * wd_ref[...] + bf_ref[...])
    out_ref[...] = jnp.tanh(h)


def _tc_head(gat, wn, wf, wd, bf):
    return pl.pallas_call(
        _tc_head_body,
        grid=(1,),
        in_specs=[
            pl.BlockSpec((2 * B, D), lambda i: (0, 0)),
            pl.BlockSpec((D, HID), lambda i: (0, 0)),
            pl.BlockSpec((D, HID), lambda i: (0, 0)),
            pl.BlockSpec((1, HID), lambda i: (0, 0)),
            pl.BlockSpec((1, HID), lambda i: (0, 0)),
        ],
        out_specs=pl.BlockSpec((B, HID), lambda i: (0, 0)),
        out_shape=jax.ShapeDtypeStruct((B, HID), jnp.float32),
    )(gat, wn, wf, wd, bf)


def kernel(x, edge_index, edge_type, nest_tensor, food_tensor, W_root, W_rel,
           b_rgcn, W_fc, b_fc):
    eshape = (N_EDGES // D, D)
    edata = _tc_pack(edge_index[0].reshape(eshape),
                     edge_index[1].reshape(eshape),
                     edge_type.astype(jnp.int32).reshape(eshape))
    edata = edata.reshape(N_EDGES)

    root = _tc_root(x, W_root, b_rgcn.reshape(1, D))
    xp = _tc_xp(x, W_rel)

    cnt_parts = _sc_count(edata)
    w_tab = _tc_weights(cnt_parts).reshape(CNT_ROWS * D)
    accs = _sc_agg(edata, xp, w_tab)
    reps = _tc_relu(root, accs)

    gidx = jnp.concatenate([nest_tensor, food_tensor]).astype(jnp.int32)
    gat = _sc_gather(reps, gidx)

    wn = W_fc[:D]
    wf = W_fc[D:2 * D]
    wd = W_fc[2 * D].reshape(1, HID)
    return _tc_head(gat, wn, wf, wd, b_fc.reshape(1, HID))


# speedup vs baseline: 1.5095x; 1.1917x over previous
import functools

import jax
import jax.numpy as jnp
from jax import lax
from jax.experimental import pallas as pl
from jax.experimental.pallas import tpu as pltpu
from jax.experimental.pallas import tpu_sc as plsc

N_NODES = 10000
N_EDGES = 320000
D = 128
R = 8
HID = 256
B = 1024

NC, NS, L = 2, 16, 16
NW = NC * NS
EPT = N_EDGES // NW
CW = 80
NCHUNK = EPT // CW
NBUF = 2
CNT_ROWS = 640
ACC_ROWS = N_NODES + L
ZPT = ACC_ROWS // NS
ROWS_PT = N_NODES // NS
GPT = 2 * B // NW

_mesh = plsc.VectorSubcoreMesh(core_axis_name="c", subcore_axis_name="s")
_sc_params = pltpu.CompilerParams(use_tc_tiling_on_sc=False,
                                  needs_layout_passes=False)


def _bcast_lane(v, e):
    idx = jnp.full((L,), e, jnp.int32)
    return lax.gather(
        v, idx[:, None],
        lax.GatherDimensionNumbers(offset_dims=(), collapsed_slice_dims=(0,),
                                   start_index_map=(0,)),
        (1,), mode=lax.GatherScatterMode.PROMISE_IN_BOUNDS)


@functools.partial(
    pl.kernel,
    out_type=jax.ShapeDtypeStruct((NW, CNT_ROWS, D), jnp.float32),
    mesh=_mesh,
    compiler_params=_sc_params,
    scratch_types=[
        pltpu.VMEM((CNT_ROWS, D), jnp.float32),
        pltpu.VMEM((EPT,), jnp.int32),
    ],
)
def _sc_count(edata, out, cnt_v, ebuf):
    wid = lax.axis_index("s") * NC + lax.axis_index("c")
    zeros = jnp.zeros((L,), jnp.float32)
    ones = jnp.ones((L,), jnp.float32)

    def zrow(i, _):
        for j in range(D // L):
            cnt_v[i, pl.ds(j * L, L)] = zeros
        return _
    lax.fori_loop(0, CNT_ROWS, zrow, None)

    pltpu.sync_copy(edata.at[pl.ds(wid * EPT, EPT)], ebuf)

    def cbody(g, _):
        ev = ebuf[pl.ds(g * L, L)]
        flat = lax.bitwise_and(ev, 0x1FFFF)
        row = lax.shift_right_logical(flat, 7)
        col = lax.bitwise_and(flat, 127)
        plsc.addupdate_scatter(cnt_v, [row, col], ones)
        return _
    lax.fori_loop(0, EPT // L, cbody, None)
    pltpu.sync_copy(cnt_v, out.at[wid])


@functools.partial(
    pl.kernel,
    out_type=jax.ShapeDtypeStruct((NC, N_NODES, D), jnp.float32),
    mesh=_mesh,
    compiler_params=_sc_params,
    scratch_types=[
        pltpu.VMEM((EPT,), jnp.int32),
        pltpu.VMEM((NBUF, CW), jnp.int32),
        pltpu.VMEM((NBUF, CW), jnp.int32),
        pltpu.VMEM((NBUF, CW), jnp.int32),
        pltpu.VMEM((CW,), jnp.int32),
        pltpu.VMEM((NBUF, CW), jnp.float32),
        pltpu.VMEM((NBUF, CW, D), jnp.float32),
        pltpu.VMEM((CW, D), jnp.float32),
        pltpu.VMEM_SHARED((ACC_ROWS, D), jnp.float32),
        pltpu.SemaphoreType.DMA,
        pltpu.SemaphoreType.DMA,
    ],
)
def _sc_agg(edata, xp, w_hbm, out, ebuf, rix, wix, six, ssix, wbuf, rows,
            stage, acc_sh, gsem, ssem):
    cid = lax.axis_index("c")
    sid = lax.axis_index("s")
    wid = sid * NC + cid
    zeros = jnp.zeros((L,), jnp.float32)
    junk = jnp.full((L,), N_NODES, jnp.int32)

    pltpu.sync_copy(edata.at[pl.ds(wid * EPT, EPT)], ebuf)

    def zrow(i, _):
        for j in range(D // L):
            stage[i, pl.ds(j * L, L)] = zeros
        return _
    lax.fori_loop(0, CW, zrow, None)
    zbase = sid * ZPT
    for k in range(ZPT // CW):
        pltpu.sync_copy(stage, acc_sh.at[pl.ds(zbase + k * CW, CW)])
    ztail = ZPT % CW
    pltpu.sync_copy(stage.at[pl.ds(0, ztail)],
                    acc_sh.at[pl.ds(zbase + (ZPT // CW) * CW, ztail)])
    plsc.subcore_barrier()

    for j in range(CW // L):
        ssix[pl.ds(j * L, L)] = junk

    def build_idx(c, t):
        valid = c < NCHUNK
        base = jnp.where(valid, c, 0) * CW
        for j in range(CW // L):
            ev = ebuf[pl.ds(base + j * L, L)]
            wix[t, pl.ds(j * L, L)] = lax.bitwise_and(ev, 0x1FFFF)
            rix[t, pl.ds(j * L, L)] = (
                lax.bitwise_and(ev, 7) * N_NODES
                + lax.shift_right_logical(ev, 17))
            s = lax.bitwise_and(lax.shift_right_logical(ev, 3), 0x3FFF)
            six[t, pl.ds(j * L, L)] = jnp.where(valid, s, junk)

    def fire_g(t):
        pltpu.async_copy(xp.at[rix.at[t]], rows.at[t], gsem)
        pltpu.async_copy(w_hbm.at[wix.at[t]], wbuf.at[t], gsem)

    def wait_g(t):
        pltpu.make_async_copy(xp.at[rix.at[t]], rows.at[t], gsem).wait()
        pltpu.make_async_copy(w_hbm.at[wix.at[t]], wbuf.at[t], gsem).wait()

    def wait_s():
        pltpu.make_async_copy(stage, acc_sh.at[ssix], ssem).wait()

    def process(t):
        wait_g(t)
        wait_s()
        for j in range(CW // L):
            ssix[pl.ds(j * L, L)] = six[t, pl.ds(j * L, L)]
        for j in range(CW // L):
            wv = wbuf[t, pl.ds(j * L, L)]
            for e in range(L):
                wb = _bcast_lane(wv, e)
                r = j * L + e
                for f in range(D // L):
                    stage[r, pl.ds(f * L, L)] = \
                        rows[t, r, pl.ds(f * L, L)] * wb
        pltpu.async_copy(stage, acc_sh.at[ssix], ssem, add=True)

    pltpu.async_copy(stage, acc_sh.at[ssix], ssem, add=True)
    for t in range(NBUF):
        build_idx(t, t)
        fire_g(t)

    def pbody(k, _):
        c0 = k * NBUF
        for t in range(NBUF):
            process(t)
            build_idx(c0 + t + NBUF, t)
            fire_g(t)
        return _
    nsteady = NCHUNK // NBUF
    ntail = NCHUNK - nsteady * NBUF
    lax.fori_loop(0, nsteady, pbody, None)
    for t in range(ntail):
        process(t)
    for t in range(ntail, NBUF):
        wait_g(t)
    wait_s()

    plsc.subcore_barrier()
    rbase = sid * ROWS_PT
    pltpu.sync_copy(acc_sh.at[pl.ds(rbase, ROWS_PT)],
                    out.at[cid, pl.ds(rbase, ROWS_PT)])


@functools.partial(
    pl.kernel,
    out_type=jax.ShapeDtypeStruct((2 * B, D), jnp.float32),
    mesh=_mesh,
    compiler_params=_sc_params,
    scratch_types=[
        pltpu.VMEM((GPT,), jnp.int32),
        pltpu.VMEM((GPT,), jnp.int32),
        pltpu.VMEM((GPT, D), jnp.float32),
        pltpu.VMEM((GPT, D), jnp.float32),
        pltpu.VMEM((GPT, D), jnp.float32),
        pltpu.SemaphoreType.DMA,
    ],
)
def _sc_gather(root, accs, idx_hbm, out, ibuf, ibuf2, r0, r1, r2, sem):
    wid = lax.axis_index("s") * NC + lax.axis_index("c")
    base = wid * GPT
    pltpu.sync_copy(idx_hbm.at[pl.ds(base, GPT)], ibuf)
    for j in range(GPT // L):
        ibuf2[pl.ds(j * L, L)] = ibuf[pl.ds(j * L, L)] + N_NODES
    d0 = pltpu.async_copy(root.at[ibuf], r0, sem)
    d1 = pltpu.async_copy(accs.at[ibuf], r1, sem)
    d2 = pltpu.async_copy(accs.at[ibuf2], r2, sem)
    d0.wait()
    d1.wait()
    d2.wait()
    def srow(i, _):
        for j in range(D // L):
            r0[i, pl.ds(j * L, L)] = (r0[i, pl.ds(j * L, L)]
                                      + r1[i, pl.ds(j * L, L)]
                                      + r2[i, pl.ds(j * L, L)])
        return _
    lax.fori_loop(0, GPT, srow, None)
    pltpu.sync_copy(r0, out.at[pl.ds(base, GPT)])


_TC_ROWS = 1000
_EB = N_EDGES // D


def _tc_proj_body(s_ref, d_ref, t_ref, x_ref, wr_ref, b_ref, wrel_ref,
                  pack_ref, root_ref, xp_ref):
    @pl.when(pl.program_id(0) == 0)
    def _():
        pack_ref[...] = ((s_ref[...] << 17) | (d_ref[...] << 3) | t_ref[...])
    xv = x_ref[...]
    root_ref[...] = jnp.dot(xv, wr_ref[...],
                            preferred_element_type=jnp.float32) + b_ref[...]
    for r in range(R):
        xp_ref[r] = jnp.dot(xv, wrel_ref[r],
                            preferred_element_type=jnp.float32)


def _tc_proj(src, dst, ty, x, w_root, b, w_rel):
    return pl.pallas_call(
        _tc_proj_body,
        grid=(N_NODES // _TC_ROWS,),
        in_specs=[
            pl.BlockSpec((_EB, D), lambda i: (0, 0)),
            pl.BlockSpec((_EB, D), lambda i: (0, 0)),
            pl.BlockSpec((_EB, D), lambda i: (0, 0)),
            pl.BlockSpec((_TC_ROWS, D), lambda i: (i, 0)),
            pl.BlockSpec((D, D), lambda i: (0, 0)),
            pl.BlockSpec((1, D), lambda i: (0, 0)),
            pl.BlockSpec((R, D, D), lambda i: (0, 0, 0)),
        ],
        out_specs=[
            pl.BlockSpec((_EB, D), lambda i: (0, 0)),
            pl.BlockSpec((_TC_ROWS, D), lambda i: (i, 0)),
            pl.BlockSpec((R, _TC_ROWS, D), lambda i: (0, i, 0)),
        ],
        out_shape=[
            jax.ShapeDtypeStruct((N_EDGES // D, D), jnp.int32),
            jax.ShapeDtypeStruct((N_NODES, D), jnp.float32),
            jax.ShapeDtypeStruct((R, N_NODES, D), jnp.float32),
        ],
    )(src, dst, ty, x, w_root, b, w_rel)


def _tc_weights_body(cnt_ref, w_ref):
    total = jnp.sum(cnt_ref[...], axis=0)
    w_ref[...] = 1.0 / jnp.maximum(total, 1.0)


def _tc_weights(cnt_parts):
    blk = CNT_ROWS // 10
    return pl.pallas_call(
        _tc_weights_body,
        grid=(10,),
        in_specs=[pl.BlockSpec((NW, blk, D), lambda i: (0, i, 0))],
        out_specs=pl.BlockSpec((blk, D), lambda i: (i, 0)),
        out_shape=jax.ShapeDtypeStruct((CNT_ROWS, D), jnp.float32),
    )(cnt_parts)


def _tc_head_body(g_ref, wn_ref, wf_ref, wd_ref, bf_ref, out_ref):
    g = jnp.maximum(g_ref[...], 0.0)
    ne = g[:B]
    fe = g[B:]
    diff = fe - ne
    dist = jnp.sqrt(jnp.sum(diff * diff, axis=1, keepdims=True))
    h = (jnp.dot(ne, wn_ref[...], preferred_element_type=jnp.float32)
         + jnp.dot(fe, wf_ref[...], preferred_element_type=jnp.float32)
         + dist * wd_ref[...] + bf_ref[...])
    out_ref[...] = jnp.tanh(h)


def _tc_head(gat, wn, wf, wd, bf):
    return pl.pallas_call(
        _tc_head_body,
        grid=(1,),
        in_specs=[
            pl.BlockSpec((2 * B, D), lambda i: (0, 0)),
            pl.BlockSpec((D, HID), lambda i: (0, 0)),
            pl.BlockSpec((D, HID), lambda i: (0, 0)),
            pl.BlockSpec((1, HID), lambda i: (0, 0)),
            pl.BlockSpec((1, HID), lambda i: (0, 0)),
        ],
        out_specs=pl.BlockSpec((B, HID), lambda i: (0, 0)),
        out_shape=jax.ShapeDtypeStruct((B, HID), jnp.float32),
    )(gat, wn, wf, wd, bf)


def kernel(x, edge_index, edge_type, nest_tensor, food_tensor, W_root, W_rel,
           b_rgcn, W_fc, b_fc):
    eshape = (N_EDGES // D, D)
    pack, root, xp3 = _tc_proj(edge_index[0].reshape(eshape),
                               edge_index[1].reshape(eshape),
                               edge_type.astype(jnp.int32).reshape(eshape),
                               x, W_root, b_rgcn.reshape(1, D), W_rel)
    edata = pack.reshape(N_EDGES)
    xp = xp3.reshape(R * N_NODES, D)

    cnt_parts = _sc_count(edata)
    w_tab = _tc_weights(cnt_parts).reshape(CNT_ROWS * D)
    accs = _sc_agg(edata, xp, w_tab)

    gidx = jnp.concatenate([nest_tensor, food_tensor]).astype(jnp.int32)
    gat = _sc_gather(root, accs.reshape(NC * N_NODES, D), gidx)

    wn = W_fc[:D]
    wf = W_fc[D:2 * D]
    wd = W_fc[2 * D].reshape(1, HID)
    return _tc_head(gat, wn, wf, wd, b_fc.reshape(1, HID))
